# Initial kernel scaffold; baseline (speedup 1.0000x reference)
import jax
import jax.numpy as jnp
from jax.experimental import pallas as pl


def kernel(x, edge_index, emb, W0, att_src0, att_dst0, b0, g0, be0, W1, att_src1, att_dst1, b1, g1, be1, W2, att_src2, att_dst2, b2):
    raise NotImplementedError("write your pallas kernel here")


# speedup vs baseline: 1.0183x; 1.0183x over previous
import functools

import jax
import jax.numpy as jnp
from jax import lax
from jax.experimental import pallas as pl

_N_BLOCK = 1000
_BN_SCALE = float((1.0 + 1e-5) ** -0.5)


def _heads_reduce(xp, att, H, F):
    cols = []
    for h in range(H):
        sl = xp[:, h * F:(h + 1) * F]
        c = lax.dot_general(sl, att[h, :], (((1,), (0,)), ((), ())),
                            preferred_element_type=jnp.float32)
        cols.append(c.reshape(-1, 1))
    return jnp.concatenate(cols, axis=1)


def _dense0_body(x_ref, emb_ref, w_ref, asrc_ref, adst_ref,
                 xp_ref, as_ref, ad_ref, *, H, F, V):
    xv = x_ref[...]
    base = xv[:, :-1]
    cell = xv[:, -1].astype(jnp.int32)
    iota = lax.broadcasted_iota(jnp.int32, (1, V), 1)
    oh = (cell[:, None] == iota).astype(jnp.float32)
    er = jnp.dot(oh, emb_ref[...], preferred_element_type=jnp.float32)
    hin = jnp.concatenate([base, er], axis=1)
    xp = lax.dot_general(hin, w_ref[...], (((1,), (1,)), ((), ())),
                         preferred_element_type=jnp.float32)
    xp_ref[...] = xp
    as_ref[...] = _heads_reduce(xp, asrc_ref[...], H, F)
    ad_ref[...] = _heads_reduce(xp, adst_ref[...], H, F)


def _dense_mid_body(prev_ref, b_ref, g_ref, be_ref, w_ref, asrc_ref, adst_ref,
                    xp_ref, as_ref, ad_ref, *, H, F):
    hv = prev_ref[...] + b_ref[...][None, :]
    scale = g_ref[...] * _BN_SCALE
    hv = hv * scale[None, :] + be_ref[...][None, :]
    hv = jnp.where(hv > 0, hv, jnp.exp(hv) - 1.0)
    xp = lax.dot_general(hv, w_ref[...], (((1,), (1,)), ((), ())),
                         preferred_element_type=jnp.float32)
    xp_ref[...] = xp
    as_ref[...] = _heads_reduce(xp, asrc_ref[...], H, F)
    ad_ref[...] = _heads_reduce(xp, adst_ref[...], H, F)


def _full(shape):
    return pl.BlockSpec(shape, lambda i: tuple(0 for _ in shape))


def _dense0(x, emb, W, asrc, adst):
    N, Din = x.shape
    H, F = asrc.shape
    V, De = emb.shape
    grid = (N // _N_BLOCK,)
    body = functools.partial(_dense0_body, H=H, F=F, V=V)
    return pl.pallas_call(
        body,
        grid=grid,
        in_specs=[
            pl.BlockSpec((_N_BLOCK, Din), lambda i: (i, 0)),
            _full((V, De)),
            _full(W.shape),
            _full(asrc.shape),
            _full(adst.shape),
        ],
        out_specs=[
            pl.BlockSpec((_N_BLOCK, H * F), lambda i: (i, 0)),
            pl.BlockSpec((_N_BLOCK, H), lambda i: (i, 0)),
            pl.BlockSpec((_N_BLOCK, H), lambda i: (i, 0)),
        ],
        out_shape=[
            jax.ShapeDtypeStruct((N, H * F), jnp.float32),
            jax.ShapeDtypeStruct((N, H), jnp.float32),
            jax.ShapeDtypeStruct((N, H), jnp.float32),
        ],
    )(x, emb, W, asrc, adst)


def _dense_mid(prev, b, g, be, W, asrc, adst):
    N, Din = prev.shape
    H, F = asrc.shape
    grid = (N // _N_BLOCK,)
    body = functools.partial(_dense_mid_body, H=H, F=F)
    return pl.pallas_call(
        body,
        grid=grid,
        in_specs=[
            pl.BlockSpec((_N_BLOCK, Din), lambda i: (i, 0)),
            _full(b.shape),
            _full(g.shape),
            _full(be.shape),
            _full(W.shape),
            _full(asrc.shape),
            _full(adst.shape),
        ],
        out_specs=[
            pl.BlockSpec((_N_BLOCK, H * F), lambda i: (i, 0)),
            pl.BlockSpec((_N_BLOCK, H), lambda i: (i, 0)),
            pl.BlockSpec((_N_BLOCK, H), lambda i: (i, 0)),
        ],
        out_shape=[
            jax.ShapeDtypeStruct((N, H * F), jnp.float32),
            jax.ShapeDtypeStruct((N, H), jnp.float32),
            jax.ShapeDtypeStruct((N, H), jnp.float32),
        ],
    )(prev, b, g, be, W, asrc, adst)


def _edge_phase(xp, a_s, a_d, src, dst, N, H, F):
    alpha = a_s[src] + a_d[dst]
    alpha = jnp.where(alpha > 0, alpha, 0.2 * alpha)
    amax = jax.ops.segment_max(alpha, dst, num_segments=N)
    amax = jnp.where(jnp.isfinite(amax), amax, 0.0)
    e = jnp.exp(alpha - amax[dst])
    denom = jax.ops.segment_sum(e, dst, num_segments=N)
    coef = e / (denom[dst] + 1e-16)
    msg = xp.reshape(N, H, F)[src] * coef[:, :, None]
    out = jax.ops.segment_sum(msg, dst, num_segments=N)
    return out.reshape(N, H * F)


def kernel(x, edge_index, emb, W0, att_src0, att_dst0, b0, g0, be0,
           W1, att_src1, att_dst1, b1, g1, be1,
           W2, att_src2, att_dst2, b2):
    N = x.shape[0]
    loop = jnp.arange(N, dtype=edge_index.dtype)
    src = jnp.concatenate([edge_index[0], loop])
    dst = jnp.concatenate([edge_index[1], loop])

    xp0, as0, ad0 = _dense0(x, emb, W0, att_src0, att_dst0)
    h0 = _edge_phase(xp0, as0, ad0, src, dst, N, *att_src0.shape)

    xp1, as1, ad1 = _dense_mid(h0, b0, g0, be0, W1, att_src1, att_dst1)
    h1 = _edge_phase(xp1, as1, ad1, src, dst, N, *att_src1.shape)

    xp2, as2, ad2 = _dense_mid(h1, b1, g1, be1, W2, att_src2, att_dst2)
    h2 = _edge_phase(xp2, as2, ad2, src, dst, N, *att_src2.shape)
    return h2 + b2[None, :]
